# baseline (speedup 1.0000x reference)
import functools

import jax
import jax.numpy as jnp
from jax import lax
from jax.experimental import pallas as pl
from jax.experimental.pallas import tpu as pltpu
from jax.experimental.pallas import tpu_sc as plsc

_B, _W, _N, _H = 16, 32, 400, 256
_BW = _B * _W
_TOPK = 10
_KPAD = 16
_EPS = 1e-5
_R1 = 64
_R2 = _W
_NC, _NS, _L = 2, 16, 16
_NW = _NC * _NS
_ROWS_PER_SUBCORE = _BW // _NW
_INV_SQRT2 = 0.7071067811865476


def _readout_body(x_ref, out_ref):
    xb = x_ref[...]
    out_ref[...] = jnp.mean(xb, axis=1) + jnp.max(xb, axis=1)


def _middle_body(xr_ref, we_ref, be_ref, g_ref, bt_ref, wa_ref, ba_ref,
                 att_ref, attT_ref, idx_ref, w_ref):
    xr = xr_ref[...]
    h = jnp.dot(xr, we_ref[...], preferred_element_type=jnp.float32) + be_ref[...]
    mu = jnp.mean(h, axis=0, keepdims=True)
    var = jnp.mean((h - mu) ** 2, axis=0, keepdims=True)
    h = (h - mu) / jnp.sqrt(var + _EPS) * g_ref[...] + bt_ref[...]
    h = 0.5 * h * (1.0 + lax.erf(h * _INV_SQRT2))
    logits = jnp.dot(h, wa_ref[...], preferred_element_type=jnp.float32) + ba_ref[...]
    att = jax.nn.sigmoid(logits)
    att_ref[...] = att
    attT_ref[...] = jnp.transpose(att.reshape(_B, _W, _N), (1, 0, 2))

    iota_n = lax.broadcasted_iota(jnp.int32, (_BW, _N), 1)
    row_base = lax.broadcasted_iota(jnp.int32, (_BW, 1), 0) * _N
    a = att
    idx_cols = []
    w_cols = []
    for _ in range(_TOPK):
        m = jnp.max(a, axis=1, keepdims=True)
        sel = jnp.min(jnp.where(a == m, iota_n, _N), axis=1, keepdims=True)
        idx_cols.append(row_base + sel)
        w_cols.append(m)
        a = jnp.where(iota_n == sel, -1.0, a)
    for _ in range(_KPAD - _TOPK):
        idx_cols.append(row_base)
        w_cols.append(jnp.zeros((_BW, 1), jnp.float32))
    idx_ref[...] = jnp.concatenate(idx_cols, axis=1)
    w_ref[...] = jnp.concatenate(w_cols, axis=1)


def _selected_sc(x_flat, idx_flat, w_flat):
    mesh = plsc.VectorSubcoreMesh(core_axis_name="c", subcore_axis_name="s")
    nidx = _ROWS_PER_SUBCORE * _KPAD
    half = nidx // 2

    @functools.partial(
        pl.kernel,
        mesh=mesh,
        out_type=jax.ShapeDtypeStruct((_BW, _H), jnp.float32),
        scratch_types=[
            pltpu.VMEM((nidx,), jnp.int32),
            pltpu.VMEM((nidx,), jnp.float32),
            pltpu.VMEM((nidx, _H), jnp.float32),
            pltpu.VMEM((_ROWS_PER_SUBCORE, _H), jnp.float32),
            pltpu.SemaphoreType.DMA,
            pltpu.SemaphoreType.DMA,
        ],
    )
    def k(x_hbm, idx_hbm, w_hbm, out_hbm, idx_v, w_v, rows_v,
          out_v, sem0, sem1):
        wid = lax.axis_index("s") * _NC + lax.axis_index("c")
        base = wid * _ROWS_PER_SUBCORE
        cpi = pltpu.async_copy(idx_hbm.at[pl.ds(base * _KPAD, nidx)], idx_v, sem0)
        cpw = pltpu.async_copy(w_hbm.at[pl.ds(base * _KPAD, nidx)], w_v, sem1)
        cpi.wait()
        cpw.wait()
        cp0 = pltpu.async_copy(
            x_hbm.at[idx_v.at[pl.ds(0, half)]], rows_v.at[pl.ds(0, half)], sem0)
        cp1 = pltpu.async_copy(
            x_hbm.at[idx_v.at[pl.ds(half, half)]], rows_v.at[pl.ds(half, half)], sem1)
        cp0.wait()
        cp1.wait()

        def body(r, carry):
            w_vec = w_v[pl.ds(r * _KPAD, _L)]
            for hh in range(_H // _L):
                acc = rows_v[r * _KPAD, pl.ds(hh * _L, _L)] * w_vec[0]
                for kk in range(1, _TOPK):
                    acc = acc + rows_v[r * _KPAD + kk, pl.ds(hh * _L, _L)] * w_vec[kk]
                out_v[r, pl.ds(hh * _L, _L)] = acc
            return carry

        lax.fori_loop(0, _ROWS_PER_SUBCORE, body, 0)
        pltpu.sync_copy(out_v, out_hbm.at[pl.ds(base, _ROWS_PER_SUBCORE)])

    return k(x_flat, idx_flat, w_flat)


def _wmean_body(att_ref, x_ref, out_ref):
    out_ref[...] = lax.dot_general(
        att_ref[...], x_ref[...],
        dimension_numbers=(((1,), (1,)), ((0,), (0,))),
        preferred_element_type=jnp.float32) * (1.0 / _N)


def kernel(x, W_embed, b_embed, bn_gamma, bn_beta, W_attend, b_attend):
    xf = x.reshape(_BW, _N, _H)

    readout = pl.pallas_call(
        _readout_body,
        grid=(_BW // _R1,),
        in_specs=[pl.BlockSpec((_R1, _N, _H), lambda i: (i, 0, 0))],
        out_specs=pl.BlockSpec((_R1, _H), lambda i: (i, 0)),
        out_shape=jax.ShapeDtypeStruct((_BW, _H), jnp.float32),
    )(xf)

    att, att_t, idx, w = pl.pallas_call(
        _middle_body,
        out_shape=(
            jax.ShapeDtypeStruct((_BW, _N), jnp.float32),
            jax.ShapeDtypeStruct((_W, _B, _N), jnp.float32),
            jax.ShapeDtypeStruct((_BW, _KPAD), jnp.int32),
            jax.ShapeDtypeStruct((_BW, _KPAD), jnp.float32),
        ),
    )(readout, W_embed, b_embed.reshape(1, _H), bn_gamma.reshape(1, _H),
      bn_beta.reshape(1, _H), W_attend, b_attend.reshape(1, _N))

    selected = _selected_sc(x.reshape(_BW * _N, _H),
                            idx.reshape(_BW * _KPAD),
                            w.reshape(_BW * _KPAD))

    output = pl.pallas_call(
        _wmean_body,
        grid=(_B,),
        in_specs=[
            pl.BlockSpec((_R2, _N), lambda i: (i, 0)),
            pl.BlockSpec((_R2, _N, _H), lambda i: (i, 0, 0)),
        ],
        out_specs=pl.BlockSpec((_R2, _H), lambda i: (i, 0)),
        out_shape=jax.ShapeDtypeStruct((_BW, _H), jnp.float32),
    )(att, xf)

    return (output.reshape(_B, _W, _H),
            selected.reshape(_B, _W, _H),
            att_t)


# speedup vs baseline: 1.0516x; 1.0516x over previous
import functools

import jax
import jax.numpy as jnp
from jax import lax
from jax.experimental import pallas as pl
from jax.experimental.pallas import tpu as pltpu
from jax.experimental.pallas import tpu_sc as plsc

_B, _W, _N, _H = 16, 32, 400, 256
_BW = _B * _W
_TOPK = 10
_KPAD = 16
_EPS = 1e-5
_R1 = 32
_R2 = _W
_NC, _NS, _L = 2, 16, 16
_NW = _NC * _NS
_ROWS_PER_SUBCORE = _BW // _NW
_INV_SQRT2 = 0.7071067811865476


def _readout_body(x_ref, out_ref):
    xb = x_ref[...]
    out_ref[...] = jnp.mean(xb, axis=1) + jnp.max(xb, axis=1)


def _middle_body(xr_ref, we_ref, be_ref, g_ref, bt_ref, wa_ref, ba_ref,
                 att_ref, attT_ref, idx_ref, w_ref):
    xr = xr_ref[...]
    h = jnp.dot(xr, we_ref[...], preferred_element_type=jnp.float32) + be_ref[...]
    mu = jnp.mean(h, axis=0, keepdims=True)
    var = jnp.mean((h - mu) ** 2, axis=0, keepdims=True)
    h = (h - mu) / jnp.sqrt(var + _EPS) * g_ref[...] + bt_ref[...]
    h = 0.5 * h * (1.0 + lax.erf(h * _INV_SQRT2))
    logits = jnp.dot(h, wa_ref[...], preferred_element_type=jnp.float32) + ba_ref[...]
    att = jax.nn.sigmoid(logits)
    att_ref[...] = att
    attT_ref[...] = jnp.transpose(att.reshape(_B, _W, _N), (1, 0, 2))

    iota_n = lax.broadcasted_iota(jnp.int32, (_BW, _N), 1)
    row_base = lax.broadcasted_iota(jnp.int32, (_BW, 1), 0) * _N
    a = att
    idx_cols = []
    w_cols = []
    for _ in range(_TOPK):
        m = jnp.max(a, axis=1, keepdims=True)
        sel = jnp.min(jnp.where(a == m, iota_n, _N), axis=1, keepdims=True)
        idx_cols.append(row_base + sel)
        w_cols.append(m)
        a = jnp.where(iota_n == sel, -1.0, a)
    for _ in range(_KPAD - _TOPK):
        idx_cols.append(row_base)
        w_cols.append(jnp.zeros((_BW, 1), jnp.float32))
    idx_ref[...] = jnp.concatenate(idx_cols, axis=1)
    w_ref[...] = jnp.concatenate(w_cols, axis=1)


def _selected_sc(x_flat, idx_flat, w_flat):
    mesh = plsc.VectorSubcoreMesh(core_axis_name="c", subcore_axis_name="s")
    nidx = _ROWS_PER_SUBCORE * _KPAD
    half = nidx // 2

    @functools.partial(
        pl.kernel,
        mesh=mesh,
        out_type=jax.ShapeDtypeStruct((_BW, _H), jnp.float32),
        scratch_types=[
            pltpu.VMEM((nidx,), jnp.int32),
            pltpu.VMEM((nidx,), jnp.float32),
            pltpu.VMEM((nidx, _H), jnp.float32),
            pltpu.VMEM((_ROWS_PER_SUBCORE, _H), jnp.float32),
            pltpu.SemaphoreType.DMA,
            pltpu.SemaphoreType.DMA,
        ],
    )
    def k(x_hbm, idx_hbm, w_hbm, out_hbm, idx_v, w_v, rows_v,
          out_v, sem0, sem1):
        wid = lax.axis_index("s") * _NC + lax.axis_index("c")
        base = wid * _ROWS_PER_SUBCORE
        cpi = pltpu.async_copy(idx_hbm.at[pl.ds(base * _KPAD, nidx)], idx_v, sem0)
        cpw = pltpu.async_copy(w_hbm.at[pl.ds(base * _KPAD, nidx)], w_v, sem1)
        cpi.wait()
        cpw.wait()
        cp0 = pltpu.async_copy(
            x_hbm.at[idx_v.at[pl.ds(0, half)]], rows_v.at[pl.ds(0, half)], sem0)
        cp1 = pltpu.async_copy(
            x_hbm.at[idx_v.at[pl.ds(half, half)]], rows_v.at[pl.ds(half, half)], sem1)
        cp0.wait()
        cp1.wait()

        def body(r, carry):
            w_vec = w_v[pl.ds(r * _KPAD, _L)]
            for hh in range(_H // _L):
                acc = rows_v[r * _KPAD, pl.ds(hh * _L, _L)] * w_vec[0]
                for kk in range(1, _TOPK):
                    acc = acc + rows_v[r * _KPAD + kk, pl.ds(hh * _L, _L)] * w_vec[kk]
                out_v[r, pl.ds(hh * _L, _L)] = acc
            return carry

        lax.fori_loop(0, _ROWS_PER_SUBCORE, body, 0)
        pltpu.sync_copy(out_v, out_hbm.at[pl.ds(base, _ROWS_PER_SUBCORE)])

    return k(x_flat, idx_flat, w_flat)


def _wmean_body(att_ref, x_ref, out_ref):
    out_ref[...] = lax.dot_general(
        att_ref[...], x_ref[...],
        dimension_numbers=(((1,), (1,)), ((0,), (0,))),
        preferred_element_type=jnp.float32) * (1.0 / _N)


def kernel(x, W_embed, b_embed, bn_gamma, bn_beta, W_attend, b_attend):
    xf = x.reshape(_BW, _N, _H)

    readout = pl.pallas_call(
        _readout_body,
        grid=(_BW // _R1,),
        in_specs=[pl.BlockSpec((_R1, _N, _H), lambda i: (i, 0, 0))],
        out_specs=pl.BlockSpec((_R1, _H), lambda i: (i, 0)),
        out_shape=jax.ShapeDtypeStruct((_BW, _H), jnp.float32),
    )(xf)

    att, att_t, idx, w = pl.pallas_call(
        _middle_body,
        out_shape=(
            jax.ShapeDtypeStruct((_BW, _N), jnp.float32),
            jax.ShapeDtypeStruct((_W, _B, _N), jnp.float32),
            jax.ShapeDtypeStruct((_BW, _KPAD), jnp.int32),
            jax.ShapeDtypeStruct((_BW, _KPAD), jnp.float32),
        ),
    )(readout, W_embed, b_embed.reshape(1, _H), bn_gamma.reshape(1, _H),
      bn_beta.reshape(1, _H), W_attend, b_attend.reshape(1, _N))

    selected = _selected_sc(x.reshape(_BW * _N, _H),
                            idx.reshape(_BW * _KPAD),
                            w.reshape(_BW * _KPAD))

    output = pl.pallas_call(
        _wmean_body,
        grid=(_B,),
        in_specs=[
            pl.BlockSpec((_R2, _N), lambda i: (i, 0)),
            pl.BlockSpec((_R2, _N, _H), lambda i: (i, 0, 0)),
        ],
        out_specs=pl.BlockSpec((_R2, _H), lambda i: (i, 0)),
        out_shape=jax.ShapeDtypeStruct((_BW, _H), jnp.float32),
    )(att, xf)

    return (output.reshape(_B, _W, _H),
            selected.reshape(_B, _W, _H),
            att_t)


# speedup vs baseline: 1.0745x; 1.0218x over previous
import functools

import jax
import jax.numpy as jnp
from jax import lax
from jax.experimental import pallas as pl
from jax.experimental.pallas import tpu as pltpu
from jax.experimental.pallas import tpu_sc as plsc

_B, _W, _N, _H = 16, 32, 400, 256
_BW = _B * _W
_TOPK = 10
_KPAD = 16
_KROW = 128
_EPS = 1e-5
_R1 = 32
_R2 = _W
_NC, _NS, _L = 2, 16, 16
_NW = _NC * _NS
_ROWS_PER_SUBCORE = _BW // _NW
_INV_SQRT2 = 0.7071067811865476


def _readout_body(x_ref, out_ref):
    xb = x_ref[...]
    out_ref[...] = jnp.mean(xb, axis=1) + jnp.max(xb, axis=1)


def _middle_body(xr_ref, we_ref, be_ref, g_ref, bt_ref, wa_ref, ba_ref,
                 att_ref, attT_ref, idx_ref, w_ref):
    xr = xr_ref[...]
    h = jnp.dot(xr, we_ref[...], preferred_element_type=jnp.float32) + be_ref[...]
    mu = jnp.mean(h, axis=0, keepdims=True)
    var = jnp.mean((h - mu) ** 2, axis=0, keepdims=True)
    h = (h - mu) / jnp.sqrt(var + _EPS) * g_ref[...] + bt_ref[...]
    h = 0.5 * h * (1.0 + lax.erf(h * _INV_SQRT2))
    logits = jnp.dot(h, wa_ref[...], preferred_element_type=jnp.float32) + ba_ref[...]
    att = jax.nn.sigmoid(logits)
    att_ref[...] = att
    attT_ref[...] = jnp.transpose(att.reshape(_B, _W, _N), (1, 0, 2))

    iota_n = lax.broadcasted_iota(jnp.int32, (_BW, _N), 1)
    row_base = lax.broadcasted_iota(jnp.int32, (_BW, 1), 0) * _N
    a = att
    idx_cols = []
    w_cols = []
    for _ in range(_TOPK):
        m = jnp.max(a, axis=1, keepdims=True)
        sel = jnp.min(jnp.where(a == m, iota_n, _N), axis=1, keepdims=True)
        idx_cols.append(row_base + sel)
        w_cols.append(m)
        a = jnp.where(iota_n == sel, -1.0, a)
    for _ in range(_KPAD - _TOPK):
        idx_cols.append(row_base)
        w_cols.append(jnp.zeros((_BW, 1), jnp.float32))
    idx_ref[...] = jnp.concatenate(
        idx_cols + [jnp.zeros((_BW, _KROW - _KPAD), jnp.int32)], axis=1)
    w_ref[...] = jnp.concatenate(
        w_cols + [jnp.zeros((_BW, _KROW - _KPAD), jnp.float32)], axis=1)


def _selected_sc(x_flat, idx_flat, w_flat):
    mesh = plsc.VectorSubcoreMesh(core_axis_name="c", subcore_axis_name="s")
    nidx = _ROWS_PER_SUBCORE * _KPAD
    nraw = _ROWS_PER_SUBCORE * _KROW
    half = nidx // 2

    @functools.partial(
        pl.kernel,
        mesh=mesh,
        out_type=jax.ShapeDtypeStruct((_BW, _H), jnp.float32),
        scratch_types=[
            pltpu.VMEM((nraw,), jnp.int32),
            pltpu.VMEM((nraw,), jnp.float32),
            pltpu.VMEM((nidx,), jnp.int32),
            pltpu.VMEM((nidx,), jnp.float32),
            pltpu.VMEM((nidx, _H), jnp.float32),
            pltpu.VMEM((_ROWS_PER_SUBCORE, _H), jnp.float32),
            pltpu.SemaphoreType.DMA,
            pltpu.SemaphoreType.DMA,
        ],
    )
    def k(x_hbm, idx_hbm, w_hbm, out_hbm, idxr_v, wr_v, idx_v, w_v, rows_v,
          out_v, sem0, sem1):
        wid = lax.axis_index("s") * _NC + lax.axis_index("c")
        base = wid * _ROWS_PER_SUBCORE
        cpi = pltpu.async_copy(idx_hbm.at[pl.ds(base * _KROW, nraw)], idxr_v, sem0)
        cpw = pltpu.async_copy(w_hbm.at[pl.ds(base * _KROW, nraw)], wr_v, sem1)
        cpi.wait()
        cpw.wait()
        for r in range(_ROWS_PER_SUBCORE):
            idx_v[pl.ds(r * _KPAD, _KPAD)] = idxr_v[pl.ds(r * _KROW, _KPAD)]
            w_v[pl.ds(r * _KPAD, _KPAD)] = wr_v[pl.ds(r * _KROW, _KPAD)]
        cp0 = pltpu.async_copy(
            x_hbm.at[idx_v.at[pl.ds(0, half)]], rows_v.at[pl.ds(0, half)], sem0)
        cp1 = pltpu.async_copy(
            x_hbm.at[idx_v.at[pl.ds(half, half)]], rows_v.at[pl.ds(half, half)], sem1)
        cp0.wait()
        cp1.wait()

        def body(r, carry):
            w_vec = w_v[pl.ds(r * _KPAD, _L)]
            for hh in range(_H // _L):
                acc = rows_v[r * _KPAD, pl.ds(hh * _L, _L)] * w_vec[0]
                for kk in range(1, _TOPK):
                    acc = acc + rows_v[r * _KPAD + kk, pl.ds(hh * _L, _L)] * w_vec[kk]
                out_v[r, pl.ds(hh * _L, _L)] = acc
            return carry

        lax.fori_loop(0, _ROWS_PER_SUBCORE, body, 0)
        pltpu.sync_copy(out_v, out_hbm.at[pl.ds(base, _ROWS_PER_SUBCORE)])

    return k(x_flat, idx_flat, w_flat)


def _wmean_body(att_ref, x_ref, out_ref):
    out_ref[...] = lax.dot_general(
        att_ref[...], x_ref[...],
        dimension_numbers=(((1,), (1,)), ((0,), (0,))),
        preferred_element_type=jnp.float32) * (1.0 / _N)


def kernel(x, W_embed, b_embed, bn_gamma, bn_beta, W_attend, b_attend):
    xf = x.reshape(_BW, _N, _H)

    readout = pl.pallas_call(
        _readout_body,
        grid=(_BW // _R1,),
        in_specs=[pl.BlockSpec((_R1, _N, _H), lambda i: (i, 0, 0))],
        out_specs=pl.BlockSpec((_R1, _H), lambda i: (i, 0)),
        out_shape=jax.ShapeDtypeStruct((_BW, _H), jnp.float32),
    )(xf)

    att, att_t, idx, w = pl.pallas_call(
        _middle_body,
        out_shape=(
            jax.ShapeDtypeStruct((_BW, _N), jnp.float32),
            jax.ShapeDtypeStruct((_W, _B, _N), jnp.float32),
            jax.ShapeDtypeStruct((_BW, _KROW), jnp.int32),
            jax.ShapeDtypeStruct((_BW, _KROW), jnp.float32),
        ),
    )(readout, W_embed, b_embed.reshape(1, _H), bn_gamma.reshape(1, _H),
      bn_beta.reshape(1, _H), W_attend, b_attend.reshape(1, _N))

    selected = _selected_sc(x.reshape(_BW * _N, _H),
                            idx.reshape(_BW * _KROW),
                            w.reshape(_BW * _KROW))

    output = pl.pallas_call(
        _wmean_body,
        grid=(_B,),
        in_specs=[
            pl.BlockSpec((_R2, _N), lambda i: (i, 0)),
            pl.BlockSpec((_R2, _N, _H), lambda i: (i, 0, 0)),
        ],
        out_specs=pl.BlockSpec((_R2, _H), lambda i: (i, 0)),
        out_shape=jax.ShapeDtypeStruct((_BW, _H), jnp.float32),
    )(att, xf)

    return (output.reshape(_B, _W, _H),
            selected.reshape(_B, _W, _H),
            att_t)
